# Initial kernel scaffold; baseline (speedup 1.0000x reference)
import jax
import jax.numpy as jnp
from jax.experimental import pallas as pl


def kernel(n_id, memory, last_update):
    raise NotImplementedError("write your pallas kernel here")


# speedup vs baseline: 1.5644x; 1.5644x over previous
import functools

import jax
import jax.numpy as jnp
from jax import lax
from jax.experimental import pallas as pl
from jax.experimental.pallas import tpu as pltpu
from jax.experimental.pallas import tpu_sc as plsc

NUM_NODES = 100000
MEMORY_DIM = 256
BATCH = 16384

NC = 2
NS = 16
NW = NC * NS
B_PER_W = BATCH // NW
CHUNK = 128
NCHUNK = B_PER_W // CHUNK

_mesh = plsc.VectorSubcoreMesh(core_axis_name="c", subcore_axis_name="s")


@functools.partial(
    pl.kernel,
    mesh=_mesh,
    out_type=(
        jax.ShapeDtypeStruct((BATCH, MEMORY_DIM), jnp.float32),
        jax.ShapeDtypeStruct((NW, NCHUNK, CHUNK), jnp.int32),
    ),
    scratch_types=[
        pltpu.VMEM((NCHUNK, CHUNK), jnp.int32),
        pltpu.VMEM((NCHUNK, CHUNK), jnp.int32),
        pltpu.VMEM((2, CHUNK, MEMORY_DIM), jnp.float32),
        pltpu.SemaphoreType.DMA,
        pltpu.SemaphoreType.DMA,
        pltpu.SemaphoreType.DMA,
    ],
)
def _sc_gather(n_id_hbm, mem_hbm, lu_hbm, mem_out, lu_out,
               idx_v, lu_v, rows_v, sem_rows, sem_lu, sem_out):
    wid = lax.axis_index("s") * NC + lax.axis_index("c")
    base = wid * B_PER_W

    pltpu.sync_copy(n_id_hbm.at[wid], idx_v)

    lu_copies = [
        pltpu.async_copy(lu_hbm.at[idx_v.at[c]], lu_v.at[c], sem_lu)
        for c in range(NCHUNK)
    ]

    gathers = [None] * NCHUNK
    writes = [None] * NCHUNK
    gathers[0] = pltpu.async_copy(mem_hbm.at[idx_v.at[0]], rows_v.at[0],
                                  sem_rows)
    for c in range(NCHUNK):
        if c + 1 < NCHUNK:
            if c >= 1:
                writes[c - 1].wait()
            gathers[c + 1] = pltpu.async_copy(
                mem_hbm.at[idx_v.at[c + 1]], rows_v.at[(c + 1) % 2], sem_rows)
        gathers[c].wait()
        writes[c] = pltpu.async_copy(
            rows_v.at[c % 2],
            mem_out.at[pl.ds(base + c * CHUNK, CHUNK)], sem_out)
    if NCHUNK >= 2:
        writes[NCHUNK - 2].wait()
    writes[NCHUNK - 1].wait()

    for cp in lu_copies:
        cp.wait()
    pltpu.sync_copy(lu_v, lu_out.at[wid])


def kernel(n_id, memory, last_update):
    n_id_r = n_id.reshape(NW, NCHUNK, CHUNK)
    mem_out, lu_out = _sc_gather(n_id_r, memory, last_update)
    return mem_out, lu_out.reshape(BATCH), jnp.zeros((), jnp.float32)
